# Initial kernel scaffold; baseline (speedup 1.0000x reference)
import jax
import jax.numpy as jnp
from jax.experimental import pallas as pl


def kernel(x, edge_knn, edge_rsphere, edge_seq, params):
    raise NotImplementedError("write your pallas kernel here")


# speedup vs baseline: 3.3411x; 3.3411x over previous
import jax
import jax.numpy as jnp
from jax import lax
from jax.experimental import pallas as pl
from jax.experimental.pallas import tpu as pltpu
from jax.experimental.pallas import tpu_sc as plsc

_N = 10000
_D = 128
_E = 320000
_NP = 10240
_K = 80
_EPS = 1e-5
_NC = 2
_NS = 16
_ROWS = _NP // _NS
_R = 400
_GRID = _N // _R
_LAYERS = 2
_PROT_LEN = 2000


def _sc_mesh():
    return plsc.VectorSubcoreMesh(core_axis_name="c", subcore_axis_name="s")


def _hist_body(ks, kd, rs, rd, ss, sd, ones1, zeros1, out,
               h0, h1, h2, h3, h4, h5, ones_v, idx_v):
    c = lax.axis_index("c")
    s = lax.axis_index("s")
    hists = (h0, h1, h2, h3, h4, h5)
    arrs = (ks, kd, rs, rd, ss, sd)

    pltpu.sync_copy(ones1, ones_v)
    for a in range(6):
        pltpu.sync_copy(zeros1, hists[a].at[pl.ds(s * _ROWS, _ROWS)])
    plsc.subcore_barrier()

    per_tile = _E // (_NC * _NS)
    for a in range(6):
        arr = arrs[a]
        hist = hists[a]

        def body(b, carry):
            off = c * (_E // _NC) + s * per_tile + b * _K
            pltpu.sync_copy(arr.at[pl.ds(off, _K)], idx_v)
            pltpu.sync_copy(ones_v, hist.at[idx_v], add=True)
            return carry

        lax.fori_loop(0, per_tile // _K, body, 0)
    plsc.subcore_barrier()

    for a in range(6):
        pltpu.sync_copy(hists[a].at[pl.ds(s * _ROWS, _ROWS)],
                        out.at[c, a, pl.ds(s * _ROWS, _ROWS)])


def _sc_histograms(ks, kd, rs, rd, ss, sd):
    kern = pl.kernel(
        _hist_body,
        out_type=jax.ShapeDtypeStruct((_NC, 6, _NP), jnp.float32),
        mesh=_sc_mesh(),
        scratch_types=[
            pltpu.VMEM_SHARED((_NP,), jnp.float32),
            pltpu.VMEM_SHARED((_NP,), jnp.float32),
            pltpu.VMEM_SHARED((_NP,), jnp.float32),
            pltpu.VMEM_SHARED((_NP,), jnp.float32),
            pltpu.VMEM_SHARED((_NP,), jnp.float32),
            pltpu.VMEM_SHARED((_NP,), jnp.float32),
            pltpu.VMEM((_K,), jnp.float32),
            pltpu.VMEM((_K,), jnp.int32),
        ],
    )
    return kern(ks, kd, rs, rd, ss, sd,
                jnp.ones((_K,), jnp.float32),
                jnp.zeros((_ROWS,), jnp.float32))


def _agg_body(zflat, ks, kd, rs, rd, ss, sd, zeros128, out,
              acc, rows_v, sidx_v, didx_v, sem):
    c = lax.axis_index("c")
    s = lax.axis_index("s")
    arrs = ((ks, kd), (rs, rd), (ss, sd))
    per_tile = _E // (_NC * _NS)

    for et in range(3):
        src_arr, dst_arr = arrs[et]

        pltpu.sync_copy(zeros128, acc.at[pl.ds(s * _ROWS, _ROWS)])
        plsc.subcore_barrier()

        def body(b, carry):
            off = c * (_E // _NC) + s * per_tile + b * _K
            pltpu.sync_copy(src_arr.at[pl.ds(off, _K)], sidx_v)
            pltpu.sync_copy(dst_arr.at[pl.ds(off, _K)], didx_v)
            if et > 0:
                for j in range(_K // 16):
                    sidx_v[pl.ds(j * 16, 16)] = (
                        sidx_v[pl.ds(j * 16, 16)] + jnp.int32(et * _N))
            pltpu.async_copy(zflat.at[sidx_v], rows_v, sem).wait()
            pltpu.sync_copy(rows_v, acc.at[didx_v], add=True)
            return carry

        lax.fori_loop(0, per_tile // _K, body, 0)
        plsc.subcore_barrier()

        pltpu.sync_copy(acc.at[pl.ds(s * _ROWS, _ROWS)],
                        out.at[c, et, pl.ds(s * _ROWS, _ROWS)])
        plsc.subcore_barrier()


def _sc_aggregate(zflat, ks, kd, rs, rd, ss, sd):
    kern = pl.kernel(
        _agg_body,
        out_type=jax.ShapeDtypeStruct((_NC, 3, _NP, _D), jnp.float32),
        mesh=_sc_mesh(),
        scratch_types=[
            pltpu.VMEM_SHARED((_NP, _D), jnp.float32),
            pltpu.VMEM((_K, _D), jnp.float32),
            pltpu.VMEM((_K,), jnp.int32),
            pltpu.VMEM((_K,), jnp.int32),
            pltpu.SemaphoreType.DMA,
        ],
    )
    return kern(zflat, ks, kd, rs, rd, ss, sd,
                jnp.zeros((_ROWS, _D), jnp.float32))


def _norms_kernel(hist):
    def body(h_ref, o_ref):
        h = h_ref[...]
        deg = h[0] + h[1]
        o_ref[...] = lax.rsqrt(jnp.maximum(deg, 1.0))

    return pl.pallas_call(
        body,
        out_shape=jax.ShapeDtypeStruct((6, _NP), jnp.float32),
    )(hist)


def _z_kernel(h, norms_t, w):
    def body(h_ref, n_ref, w_ref, z_ref):
        hb = h_ref[...]
        for et in range(3):
            ns = n_ref[:, 2 * et:2 * et + 1]
            z_ref[et] = jnp.dot(hb * ns, w_ref[et],
                                precision=lax.Precision.HIGHEST,
                                preferred_element_type=jnp.float32)

    return pl.pallas_call(
        body,
        grid=(_GRID,),
        in_specs=[
            pl.BlockSpec((_R, _D), lambda i: (i, 0)),
            pl.BlockSpec((_R, 6), lambda i: (i, 0)),
            pl.BlockSpec((3, _D, _D), lambda i: (0, 0, 0)),
        ],
        out_specs=pl.BlockSpec((3, _R, _D), lambda i: (0, i, 0)),
        out_shape=jax.ShapeDtypeStruct((3, _N, _D), jnp.float32),
    )(h, norms_t, w)


def _combine_kernel(acc, norms_t, bsum):
    def body(a_ref, n_ref, b_ref, o_ref, st_ref):
        i = pl.program_id(0)
        a = a_ref[...]
        tot = jnp.zeros((_R, _D), jnp.float32)
        for et in range(3):
            nd = n_ref[:, 2 * et + 1:2 * et + 2]
            tot = tot + (a[0, et] + a[1, et]) * nd
        tot = tot + b_ref[...]
        o_ref[...] = tot
        colsum = jnp.sum(tot, axis=0)
        colsq = jnp.sum(tot * tot, axis=0)
        upd = jnp.concatenate(
            [colsum[None], colsq[None], jnp.zeros((6, _D), jnp.float32)], 0)

        @pl.when(i == 0)
        def _():
            st_ref[...] = upd

        @pl.when(i != 0)
        def _():
            st_ref[...] = st_ref[...] + upd

    return pl.pallas_call(
        body,
        grid=(_GRID,),
        in_specs=[
            pl.BlockSpec((_NC, 3, _R, _D), lambda i: (0, 0, i, 0)),
            pl.BlockSpec((_R, 6), lambda i: (i, 0)),
            pl.BlockSpec((1, _D), lambda i: (0, 0)),
        ],
        out_specs=[
            pl.BlockSpec((_R, _D), lambda i: (i, 0)),
            pl.BlockSpec((8, _D), lambda i: (0, 0)),
        ],
        out_shape=[
            jax.ShapeDtypeStruct((_N, _D), jnp.float32),
            jax.ShapeDtypeStruct((8, _D), jnp.float32),
        ],
    )(acc, norms_t, bsum)


def _bnfc_kernel(x, stats, gamma, beta, fcw, fcb):
    def body(x_ref, st_ref, g_ref, be_ref, w_ref, b_ref, o_ref):
        xb = x_ref[...]
        mean = st_ref[0:1, :] * (1.0 / _N)
        ex2 = st_ref[1:2, :] * (1.0 / _N)
        var = ex2 - mean * mean
        xn = (xb - mean) * lax.rsqrt(var + _EPS) * g_ref[...] + be_ref[...]
        r = jnp.maximum(xn, 0.0)
        o_ref[...] = jnp.dot(r, w_ref[...],
                             precision=lax.Precision.HIGHEST,
                             preferred_element_type=jnp.float32) + b_ref[...]

    return pl.pallas_call(
        body,
        grid=(_GRID,),
        in_specs=[
            pl.BlockSpec((_R, _D), lambda i: (i, 0)),
            pl.BlockSpec((8, _D), lambda i: (0, 0)),
            pl.BlockSpec((1, _D), lambda i: (0, 0)),
            pl.BlockSpec((1, _D), lambda i: (0, 0)),
            pl.BlockSpec((_D, _D), lambda i: (0, 0)),
            pl.BlockSpec((1, _D), lambda i: (0, 0)),
        ],
        out_specs=pl.BlockSpec((_R, _D), lambda i: (i, 0)),
        out_shape=jax.ShapeDtypeStruct((_N, _D), jnp.float32),
    )(x, stats, gamma, beta, fcw, fcb)


def kernel(x, edge_knn, edge_rsphere, edge_seq, params):
    ek = edge_knn.astype(jnp.int32)
    er = edge_rsphere.astype(jnp.int32)
    es = edge_seq.astype(jnp.int32)
    ks, kd = ek[0], ek[1]
    rs, rd = er[0], er[1]
    ss, sd = es[0], es[1]

    hist = _sc_histograms(ks, kd, rs, rd, ss, sd)
    norms_t = _norms_kernel(hist).T

    h = x
    for i in range(_LAYERS):
        lp = params["layer%d" % i]
        w = jnp.stack([lp["knn_W"], lp["rsphere_W"], lp["seq_W"]])
        bsum = (lp["knn_b"] + lp["rsphere_b"] + lp["seq_b"]).reshape(1, _D)
        z = _z_kernel(h, norms_t, w)
        zflat = z.reshape(3 * _N, _D)
        acc = _sc_aggregate(zflat, ks, kd, rs, rd, ss, sd)
        out, stats = _combine_kernel(acc, norms_t, bsum)
        h = _bnfc_kernel(out, stats,
                         lp["bn_gamma"].reshape(1, _D),
                         lp["bn_beta"].reshape(1, _D),
                         lp["fc_W"],
                         lp["fc_b"].reshape(1, _D))
    return h.reshape(-1, _PROT_LEN, _D)


# speedup vs baseline: 6.7664x; 2.0252x over previous
import jax
import jax.numpy as jnp
from jax import lax
from jax.experimental import pallas as pl
from jax.experimental.pallas import tpu as pltpu
from jax.experimental.pallas import tpu_sc as plsc

_N = 10000
_D = 128
_E = 320000
_NP = 10240
_K = 80
_EPS = 1e-5
_NC = 2
_NS = 16
_ROWS = _NP // _NS
_R = 400
_GRID = _N // _R
_LAYERS = 2
_PROT_LEN = 2000


def _sc_mesh():
    return plsc.VectorSubcoreMesh(core_axis_name="c", subcore_axis_name="s")


_PT = _E // (_NC * _NS)
_NB = _PT // _K


def _hist_body(ks, kd, rs, rd, ss, sd, ones1, zeros1, out,
               h0, h1, h2, h3, h4, h5, ones_v,
               i0, i1, i2, i3, i4, i5):
    c = lax.axis_index("c")
    s = lax.axis_index("s")
    hists = (h0, h1, h2, h3, h4, h5)
    arrs = (ks, kd, rs, rd, ss, sd)
    idxs = (i0, i1, i2, i3, i4, i5)

    pltpu.sync_copy(ones1, ones_v)
    base = c * (_E // _NC) + s * _PT
    for a in range(6):
        pltpu.sync_copy(zeros1, hists[a].at[pl.ds(s * _ROWS, _ROWS)])
        pltpu.sync_copy(arrs[a].at[pl.ds(base, _PT)], idxs[a])
    plsc.subcore_barrier()

    def body(b, carry):
        for a in range(6):
            pltpu.sync_copy(
                ones_v, hists[a].at[idxs[a].at[pl.ds(b * _K, _K)]],
                add=True)
        return carry

    lax.fori_loop(0, _NB, body, 0)
    plsc.subcore_barrier()

    for a in range(6):
        pltpu.sync_copy(hists[a].at[pl.ds(s * _ROWS, _ROWS)],
                        out.at[c, a, pl.ds(s * _ROWS, _ROWS)])


def _sc_histograms(ks, kd, rs, rd, ss, sd):
    kern = pl.kernel(
        _hist_body,
        out_type=jax.ShapeDtypeStruct((_NC, 6, _NP), jnp.float32),
        mesh=_sc_mesh(),
        scratch_types=[
            pltpu.VMEM_SHARED((_NP,), jnp.float32),
            pltpu.VMEM_SHARED((_NP,), jnp.float32),
            pltpu.VMEM_SHARED((_NP,), jnp.float32),
            pltpu.VMEM_SHARED((_NP,), jnp.float32),
            pltpu.VMEM_SHARED((_NP,), jnp.float32),
            pltpu.VMEM_SHARED((_NP,), jnp.float32),
            pltpu.VMEM((_K,), jnp.float32),
            pltpu.VMEM((_PT,), jnp.int32),
            pltpu.VMEM((_PT,), jnp.int32),
            pltpu.VMEM((_PT,), jnp.int32),
            pltpu.VMEM((_PT,), jnp.int32),
            pltpu.VMEM((_PT,), jnp.int32),
            pltpu.VMEM((_PT,), jnp.int32),
        ],
    )
    return kern(ks, kd, rs, rd, ss, sd,
                jnp.ones((_K,), jnp.float32),
                jnp.zeros((_ROWS,), jnp.float32))


def _agg_body(z3, ks, kd, rs, rd, ss, sd, zeros128, out,
              acc, r0, r1, sidx, didx, gsem0, gsem1, ssem0, ssem1):
    c = lax.axis_index("c")
    s = lax.axis_index("s")
    arrs = ((ks, kd), (rs, rd), (ss, sd))

    for et in range(3):
        src_arr, dst_arr = arrs[et]
        zc = z3.at[et]

        def gather(b, buf, sem):
            pltpu.async_copy(zc.at[sidx.at[pl.ds(b * _K, _K)]], buf, sem)

        def scatter(b, buf, sem):
            pltpu.async_copy(buf, acc.at[didx.at[pl.ds(b * _K, _K)]],
                             sem, add=True)

        def drain(sem, buf):
            pltpu.make_async_copy(zc.at[pl.ds(0, _K)], buf, sem).wait()

        pltpu.sync_copy(zeros128, acc.at[pl.ds(s * _ROWS, _ROWS)])
        base = c * (_E // _NC) + s * _PT
        pltpu.sync_copy(src_arr.at[pl.ds(base, _PT)], sidx)
        pltpu.sync_copy(dst_arr.at[pl.ds(base, _PT)], didx)
        plsc.subcore_barrier()

        gather(0, r0, gsem0)
        gather(1, r1, gsem1)

        def outer(i, carry):
            a = 2 * i
            drain(gsem0, r0)
            scatter(a, r0, ssem0)
            drain(gsem1, r1)
            scatter(a + 1, r1, ssem1)
            drain(ssem0, r0)

            @pl.when(a + 2 < _NB)
            def _():
                gather(a + 2, r0, gsem0)

            drain(ssem1, r1)

            @pl.when(a + 3 < _NB)
            def _():
                gather(a + 3, r1, gsem1)

            return carry

        lax.fori_loop(0, _NB // 2, outer, 0)
        drain(gsem0, r0)
        scatter(_NB - 1, r0, ssem0)
        drain(ssem0, r0)
        plsc.subcore_barrier()

        pltpu.sync_copy(acc.at[pl.ds(s * _ROWS, _ROWS)],
                        out.at[c, et, pl.ds(s * _ROWS, _ROWS)])
        plsc.subcore_barrier()


def _sc_aggregate(z3, ks, kd, rs, rd, ss, sd):
    kern = pl.kernel(
        _agg_body,
        out_type=jax.ShapeDtypeStruct((_NC, 3, _NP, _D), jnp.float32),
        mesh=_sc_mesh(),
        scratch_types=[
            pltpu.VMEM_SHARED((_NP, _D), jnp.float32),
            pltpu.VMEM((_K, _D), jnp.float32),
            pltpu.VMEM((_K, _D), jnp.float32),
            pltpu.VMEM((_PT,), jnp.int32),
            pltpu.VMEM((_PT,), jnp.int32),
            pltpu.SemaphoreType.DMA,
            pltpu.SemaphoreType.DMA,
            pltpu.SemaphoreType.DMA,
            pltpu.SemaphoreType.DMA,
        ],
    )
    return kern(z3, ks, kd, rs, rd, ss, sd,
                jnp.zeros((_ROWS, _D), jnp.float32))


def _norms_kernel(hist):
    def body(h_ref, o_ref):
        h = h_ref[...]
        deg = h[0] + h[1]
        o_ref[...] = lax.rsqrt(jnp.maximum(deg, 1.0))

    return pl.pallas_call(
        body,
        out_shape=jax.ShapeDtypeStruct((6, _NP), jnp.float32),
    )(hist)


def _z_kernel(h, norms_t, w):
    def body(h_ref, n_ref, w_ref, z_ref):
        hb = h_ref[...]
        for et in range(3):
            ns = n_ref[:, 2 * et:2 * et + 1]
            z_ref[et] = jnp.dot(hb * ns, w_ref[et],
                                precision=lax.Precision.HIGHEST,
                                preferred_element_type=jnp.float32)

    return pl.pallas_call(
        body,
        grid=(_GRID,),
        in_specs=[
            pl.BlockSpec((_R, _D), lambda i: (i, 0)),
            pl.BlockSpec((_R, 6), lambda i: (i, 0)),
            pl.BlockSpec((3, _D, _D), lambda i: (0, 0, 0)),
        ],
        out_specs=pl.BlockSpec((3, _R, _D), lambda i: (0, i, 0)),
        out_shape=jax.ShapeDtypeStruct((3, _N, _D), jnp.float32),
    )(h, norms_t, w)


def _combine_kernel(acc, norms_t, bsum):
    def body(a_ref, n_ref, b_ref, o_ref, st_ref):
        i = pl.program_id(0)
        a = a_ref[...]
        tot = jnp.zeros((_R, _D), jnp.float32)
        for et in range(3):
            nd = n_ref[:, 2 * et + 1:2 * et + 2]
            tot = tot + (a[0, et] + a[1, et]) * nd
        tot = tot + b_ref[...]
        o_ref[...] = tot
        colsum = jnp.sum(tot, axis=0)
        colsq = jnp.sum(tot * tot, axis=0)
        upd = jnp.concatenate(
            [colsum[None], colsq[None], jnp.zeros((6, _D), jnp.float32)], 0)

        @pl.when(i == 0)
        def _():
            st_ref[...] = upd

        @pl.when(i != 0)
        def _():
            st_ref[...] = st_ref[...] + upd

    return pl.pallas_call(
        body,
        grid=(_GRID,),
        in_specs=[
            pl.BlockSpec((_NC, 3, _R, _D), lambda i: (0, 0, i, 0)),
            pl.BlockSpec((_R, 6), lambda i: (i, 0)),
            pl.BlockSpec((1, _D), lambda i: (0, 0)),
        ],
        out_specs=[
            pl.BlockSpec((_R, _D), lambda i: (i, 0)),
            pl.BlockSpec((8, _D), lambda i: (0, 0)),
        ],
        out_shape=[
            jax.ShapeDtypeStruct((_N, _D), jnp.float32),
            jax.ShapeDtypeStruct((8, _D), jnp.float32),
        ],
    )(acc, norms_t, bsum)


def _bnfc_kernel(x, stats, gamma, beta, fcw, fcb):
    def body(x_ref, st_ref, g_ref, be_ref, w_ref, b_ref, o_ref):
        xb = x_ref[...]
        mean = st_ref[0:1, :] * (1.0 / _N)
        ex2 = st_ref[1:2, :] * (1.0 / _N)
        var = ex2 - mean * mean
        xn = (xb - mean) * lax.rsqrt(var + _EPS) * g_ref[...] + be_ref[...]
        r = jnp.maximum(xn, 0.0)
        o_ref[...] = jnp.dot(r, w_ref[...],
                             precision=lax.Precision.HIGHEST,
                             preferred_element_type=jnp.float32) + b_ref[...]

    return pl.pallas_call(
        body,
        grid=(_GRID,),
        in_specs=[
            pl.BlockSpec((_R, _D), lambda i: (i, 0)),
            pl.BlockSpec((8, _D), lambda i: (0, 0)),
            pl.BlockSpec((1, _D), lambda i: (0, 0)),
            pl.BlockSpec((1, _D), lambda i: (0, 0)),
            pl.BlockSpec((_D, _D), lambda i: (0, 0)),
            pl.BlockSpec((1, _D), lambda i: (0, 0)),
        ],
        out_specs=pl.BlockSpec((_R, _D), lambda i: (i, 0)),
        out_shape=jax.ShapeDtypeStruct((_N, _D), jnp.float32),
    )(x, stats, gamma, beta, fcw, fcb)


def kernel(x, edge_knn, edge_rsphere, edge_seq, params):
    ek = edge_knn.astype(jnp.int32)
    er = edge_rsphere.astype(jnp.int32)
    es = edge_seq.astype(jnp.int32)
    ks, kd = ek[0], ek[1]
    rs, rd = er[0], er[1]
    ss, sd = es[0], es[1]

    hist = _sc_histograms(ks, kd, rs, rd, ss, sd)
    norms_t = _norms_kernel(hist).T

    h = x
    for i in range(_LAYERS):
        lp = params["layer%d" % i]
        w = jnp.stack([lp["knn_W"], lp["rsphere_W"], lp["seq_W"]])
        bsum = (lp["knn_b"] + lp["rsphere_b"] + lp["seq_b"]).reshape(1, _D)
        z = _z_kernel(h, norms_t, w)
        acc = _sc_aggregate(z, ks, kd, rs, rd, ss, sd)
        out, stats = _combine_kernel(acc, norms_t, bsum)
        h = _bnfc_kernel(out, stats,
                         lp["bn_gamma"].reshape(1, _D),
                         lp["bn_beta"].reshape(1, _D),
                         lp["fc_W"],
                         lp["fc_b"].reshape(1, _D))
    return h.reshape(-1, _PROT_LEN, _D)
